# Initial kernel scaffold; baseline (speedup 1.0000x reference)
import jax
import jax.numpy as jnp
from jax.experimental import pallas as pl


def kernel(x, Wg, We, be):
    raise NotImplementedError("write your pallas kernel here")


# speedup vs baseline: 1.2256x; 1.2256x over previous
import functools
import jax
import jax.numpy as jnp
from jax.experimental import pallas as pl
from jax.experimental.pallas import tpu as pltpu

_E = 8
_TOPK = 2

_TM = 1024
_TO = 512


def _dense_moe_body(x_ref, wg_ref, we_ref, be_ref, out_ref, wd_ref):
    ot = pl.program_id(1)
    e = pl.program_id(2)

    @pl.when(jnp.logical_and(ot == 0, e == 0))
    def _gate():
        xb = x_ref[...]
        logits = jax.lax.dot_general(
            xb, wg_ref[...], (((1,), (1,)), ((), ())),
            preferred_element_type=jnp.float32)
        m = jnp.max(logits, axis=1, keepdims=True)
        ex = jnp.exp(logits - m)
        p = ex / jnp.sum(ex, axis=1, keepdims=True)
        lanes = jax.lax.broadcasted_iota(jnp.int32, p.shape, 1)
        m1 = jnp.max(p, axis=1, keepdims=True)
        i1 = jnp.min(jnp.where(p == m1, lanes, _E), axis=1, keepdims=True)
        p2 = jnp.where(lanes == i1, -jnp.inf, p)
        m2 = jnp.max(p2, axis=1, keepdims=True)
        i2 = jnp.min(jnp.where(p2 == m2, lanes, _E), axis=1, keepdims=True)
        wd = jnp.where(lanes == i1, m1, 0.0) + jnp.where(lanes == i2, m2, 0.0)
        wd_ref[...] = wd

    @pl.when(e == 0)
    def _zero():
        out_ref[...] = jnp.zeros_like(out_ref)

    lanes = jax.lax.broadcasted_iota(jnp.int32, wd_ref.shape, 1)
    w_e = jnp.sum(jnp.where(lanes == e, wd_ref[...], 0.0), axis=1,
                  keepdims=True)
    acc = jax.lax.dot_general(
        x_ref[...], we_ref[0], (((1,), (1,)), ((), ())),
        preferred_element_type=jnp.float32)
    acc = acc + be_ref[0]
    out_ref[...] += w_e * acc


def _dense_moe(xf, Wg, We, be):
    n_tok, in_dim = xf.shape
    out_dim = We.shape[1]
    grid = (n_tok // _TM, out_dim // _TO, _E)
    return pl.pallas_call(
        _dense_moe_body,
        grid=grid,
        in_specs=[
            pl.BlockSpec((_TM, in_dim), lambda mt, ot, e: (mt, 0)),
            pl.BlockSpec((_E, in_dim), lambda mt, ot, e: (0, 0)),
            pl.BlockSpec((1, _TO, in_dim), lambda mt, ot, e: (e, ot, 0)),
            pl.BlockSpec((1, 1, _TO), lambda mt, ot, e: (e, 0, ot)),
        ],
        out_specs=pl.BlockSpec((_TM, _TO), lambda mt, ot, e: (mt, ot)),
        out_shape=jax.ShapeDtypeStruct((n_tok, out_dim), jnp.float32),
        scratch_shapes=[pltpu.VMEM((_TM, _E), jnp.float32)],
    )(xf, Wg, We, be)


def kernel(x, Wg, We, be):
    B, in_len, n_vars = x.shape
    xf = jnp.transpose(x, (0, 2, 1)).reshape(B * n_vars, in_len)
    be3 = be.reshape(_E, 1, be.shape[-1])
    out = _dense_moe(xf, Wg, We, be3)
    out_dim = We.shape[1]
    return jnp.transpose(out.reshape(B, n_vars, out_dim), (0, 2, 1))
